# baseline (speedup 1.0000x reference)
import functools

import jax
import jax.numpy as jnp
from jax import lax
from jax.experimental import pallas as pl
from jax.experimental.pallas import tpu as pltpu
from jax.experimental.pallas import tpu_sc as plsc

_B, _N, _K = 4, 1024, 16
_E, _IN, _HID = 4, 512, 512
_NODES = _B * _N
_NH = _NODES // 2
_NPJ = 16
_NJOBS = 4
_RPD = 32
_SPJ = _NPJ * _K // _RPD
_NST = _NJOBS * _SPJ
_ACCR = _E * _NPJ
_RPT = _NH * _K // 32


def _sc_agg_body(x_hbm, idx_hbm, basev_hbm, agg_hbm, idx_v, basev_v,
                 buf, acc, gsem, osem):
    c = lax.axis_index("c")
    s = lax.axis_index("s")
    t = s * 2 + c

    pltpu.sync_copy(idx_hbm.at[pl.ds(pl.multiple_of(t * _RPT, _RPT), _RPT)],
                    idx_v)
    pltpu.sync_copy(
        basev_hbm.at[pl.ds(pl.multiple_of(t * _RPT * 16, _RPT * 16),
                           _RPT * 16)], basev_v)
    pltpu.async_copy(x_hbm.at[idx_v.at[pl.ds(0, _RPD)]], buf.at[0], gsem)

    def _outs(j, wait):
        n0g = (t * _NJOBS + j) * _NPJ
        b = n0g // _N
        nb = n0g % _N
        for e in range(_E):
            dest = pl.multiple_of((b * (_E * _N) + e * _N + nb) * _IN,
                                  _NPJ * _IN)
            cp = pltpu.make_async_copy(
                acc.at[pl.ds(e * _NPJ * _IN, _NPJ * _IN)],
                agg_hbm.at[pl.ds(dest, _NPJ * _IN)], osem)
            if wait:
                cp.wait()
            else:
                cp.start()

    def _step(st, carry):
        sl = st % 2

        @pl.when(st + 1 < _NST)
        def _():
            pltpu.async_copy(
                x_hbm.at[idx_v.at[pl.ds((st + 1) * _RPD, _RPD)]],
                buf.at[1 - sl], gsem)

        @pl.when((st % _SPJ == 0) & (st > 0))
        def _():
            _outs(st // _SPJ - 1, True)

        @pl.when(st % _SPJ == 0)
        def _():
            @plsc.parallel_loop(0, _ACCR, 1, unroll=4)
            def _zero(r):
                for cch in range(_IN // 16):
                    acc[pl.ds(r * _IN + cch * 16, 16)] = jnp.zeros(
                        (16,), jnp.float32)

        pltpu.make_async_copy(x_hbm.at[idx_v.at[pl.ds(st * _RPD, _RPD)]],
                              buf.at[sl], gsem).wait()

        @plsc.parallel_loop(0, _RPD, 1, unroll=6)
        def _row(r):
            ivec = basev_v[pl.ds((st * _RPD + r) * 16, 16)]
            for cch in range(_IN // 16):
                plsc.addupdate_scatter(
                    acc, [ivec + (cch * 16)],
                    buf[sl, r, pl.ds(cch * 16, 16)])

        @pl.when(st % _SPJ == _SPJ - 1)
        def _():
            _outs(st // _SPJ, False)
        return carry

    lax.fori_loop(0, _NST, _step, 0)
    _outs(_NJOBS - 1, True)


_sc_agg = functools.partial(
    pl.kernel,
    out_type=jax.ShapeDtypeStruct((_NH * _E * _IN,), jnp.float32),
    mesh=plsc.VectorSubcoreMesh(core_axis_name="c", subcore_axis_name="s"),
    compiler_params=pltpu.CompilerParams(needs_layout_passes=False),
    scratch_types=[
        pltpu.VMEM((_RPT,), jnp.int32),
        pltpu.VMEM((_RPT * 16,), jnp.int32),
        pltpu.VMEM((2, _RPD, _IN), jnp.float32),
        pltpu.VMEM((_ACCR * _IN,), jnp.float32),
        pltpu.SemaphoreType.DMA,
        pltpu.SemaphoreType.DMA,
    ],
)(_sc_agg_body)


def _m_body(w_ref, we_ref, m_ref):
    m_ref[0] = lax.dot_general(w_ref[...], we_ref[0], (((1,), (1,)), ((), ())),
                               preferred_element_type=jnp.float32)


def _fuse_weights(W, W_edge):
    return pl.pallas_call(
        _m_body,
        grid=(_E,),
        in_specs=[pl.BlockSpec((_IN, _HID), lambda e: (0, 0)),
                  pl.BlockSpec((1, _HID, _HID), lambda e: (e, 0, 0))],
        out_specs=pl.BlockSpec((1, _IN, _HID), lambda e: (e, 0, 0)),
        out_shape=jax.ShapeDtypeStruct((_E, _IN, _HID), jnp.float32),
    )(W, W_edge)


def _out_body(agg_ref, m_ref, bias_ref, o_ref):
    e = pl.program_id(1)
    part = jnp.dot(agg_ref[0, 0], m_ref[0], preferred_element_type=jnp.float32)

    @pl.when(e == 0)
    def _():
        o_ref[0] = part

    @pl.when(e != 0)
    def _():
        o_ref[0] += part

    @pl.when(e == _E - 1)
    def _():
        o_ref[0] = jnp.maximum(o_ref[0] * (1.0 / _K) + bias_ref[...], 0.0)


def _finalize(agg, M, bias2d):
    nb = agg.shape[0]
    return pl.pallas_call(
        _out_body,
        grid=(nb, _E),
        in_specs=[pl.BlockSpec((1, 1, _N, _IN), lambda b, e: (b, e, 0, 0)),
                  pl.BlockSpec((1, _IN, _HID), lambda b, e: (e, 0, 0)),
                  pl.BlockSpec((1, _HID), lambda b, e: (0, 0))],
        out_specs=pl.BlockSpec((1, _N, _HID), lambda b, e: (b, 0, 0)),
        out_shape=jax.ShapeDtypeStruct((nb, _N, _HID), jnp.float32),
        compiler_params=pltpu.CompilerParams(
            dimension_semantics=("arbitrary", "arbitrary")),
    )(agg, M, bias2d)


def kernel(nodes_embed, node_nei, node_edge, W, W_edge, bias):
    x_flat = nodes_embed.reshape(_NODES, _IN)
    b_idx = jnp.arange(_B, dtype=jnp.int32)[:, None, None]
    gidx = (node_nei + b_idx * _N).reshape(_NODES * _K)
    n_local = (jnp.arange(_N, dtype=jnp.int32) % _NPJ)[None, :, None]
    drow = (node_edge * _NPJ + n_local).reshape(_NODES * _K)
    basev = (drow[:, None] * _IN
             + jnp.arange(16, dtype=jnp.int32)[None, :]).reshape(-1)
    M = _fuse_weights(W, W_edge)
    bias2 = bias.reshape(1, _HID)
    hk = _NH * _K
    aggs = [_sc_agg(x_flat, gidx[h * hk:(h + 1) * hk],
                    basev[h * hk * 16:(h + 1) * hk * 16])
            for h in range(2)]
    outs = [_finalize(a.reshape(2, _E, _N, _IN), M, bias2) for a in aggs]
    return jnp.concatenate(outs, axis=0)


# speedup vs baseline: 1.2699x; 1.2699x over previous
import functools

import jax
import jax.numpy as jnp
from jax import lax
from jax.experimental import pallas as pl
from jax.experimental.pallas import tpu as pltpu
from jax.experimental.pallas import tpu_sc as plsc

_B, _N, _K = 4, 1024, 16
_E, _IN, _HID = 4, 512, 512
_NODES = _B * _N
_NH = _NODES // 2
_NPJ = 16
_NJOBS = 4
_RPD = 32
_SPJ = _NPJ * _K // _RPD
_NST = _NJOBS * _SPJ
_ACCR = _E * _NPJ
_RPT = _NH * _K // 32


def _sc_agg_body(x_hbm, idx_hbm, basev_hbm, agg_hbm, idx_v, basev_v,
                 buf, acc, gsem, osem):
    c = lax.axis_index("c")
    s = lax.axis_index("s")
    t = s * 2 + c

    pltpu.sync_copy(idx_hbm.at[pl.ds(pl.multiple_of(t * _RPT, _RPT), _RPT)],
                    idx_v)
    pltpu.sync_copy(
        basev_hbm.at[pl.ds(pl.multiple_of(t * _RPT * 16, _RPT * 16),
                           _RPT * 16)], basev_v)
    pltpu.async_copy(x_hbm.at[idx_v.at[pl.ds(0, _RPD)]], buf.at[0], gsem)

    def _outs(j, wait):
        n0g = (t * _NJOBS + j) * _NPJ
        b = n0g // _N
        nb = n0g % _N
        for e in range(_E):
            dest = pl.multiple_of((b * (_E * _N) + e * _N + nb) * _IN,
                                  _NPJ * _IN)
            cp = pltpu.make_async_copy(
                acc.at[pl.ds(e * _NPJ * _IN, _NPJ * _IN)],
                agg_hbm.at[pl.ds(dest, _NPJ * _IN)], osem)
            if wait:
                cp.wait()
            else:
                cp.start()

    def _step(st, carry):
        sl = st % 2

        @pl.when(st + 1 < _NST)
        def _():
            pltpu.async_copy(
                x_hbm.at[idx_v.at[pl.ds((st + 1) * _RPD, _RPD)]],
                buf.at[1 - sl], gsem)

        @pl.when((st % _SPJ == 0) & (st > 0))
        def _():
            _outs(st // _SPJ - 1, True)

        @pl.when(st % _SPJ == 0)
        def _():
            @plsc.parallel_loop(0, _ACCR, 1, unroll=4)
            def _zero(r):
                for cch in range(_IN // 16):
                    acc[pl.ds(r * _IN + cch * 16, 16)] = jnp.zeros(
                        (16,), jnp.float32)

        pltpu.make_async_copy(x_hbm.at[idx_v.at[pl.ds(st * _RPD, _RPD)]],
                              buf.at[sl], gsem).wait()

        @plsc.parallel_loop(0, _RPD, 1, unroll=4)
        def _row(r):
            ivec = basev_v[pl.ds((st * _RPD + r) * 16, 16)]
            for cch in range(_IN // 16):
                plsc.addupdate_scatter(
                    acc, [ivec + (cch * 16)],
                    buf[sl, r, pl.ds(cch * 16, 16)])

        @pl.when(st % _SPJ == _SPJ - 1)
        def _():
            _outs(st // _SPJ, False)
        return carry

    lax.fori_loop(0, _NST, _step, 0)
    _outs(_NJOBS - 1, True)


_sc_agg = functools.partial(
    pl.kernel,
    out_type=jax.ShapeDtypeStruct((_NH * _E * _IN,), jnp.float32),
    mesh=plsc.VectorSubcoreMesh(core_axis_name="c", subcore_axis_name="s"),
    compiler_params=pltpu.CompilerParams(needs_layout_passes=False),
    scratch_types=[
        pltpu.VMEM((_RPT,), jnp.int32),
        pltpu.VMEM((_RPT * 16,), jnp.int32),
        pltpu.VMEM((2, _RPD, _IN), jnp.float32),
        pltpu.VMEM((_ACCR * _IN,), jnp.float32),
        pltpu.SemaphoreType.DMA,
        pltpu.SemaphoreType.DMA,
    ],
)(_sc_agg_body)


def _m_body(w_ref, we_ref, m_ref):
    m_ref[0] = lax.dot_general(w_ref[...], we_ref[0], (((1,), (1,)), ((), ())),
                               preferred_element_type=jnp.float32)


def _fuse_weights(W, W_edge):
    return pl.pallas_call(
        _m_body,
        grid=(_E,),
        in_specs=[pl.BlockSpec((_IN, _HID), lambda e: (0, 0)),
                  pl.BlockSpec((1, _HID, _HID), lambda e: (e, 0, 0))],
        out_specs=pl.BlockSpec((1, _IN, _HID), lambda e: (e, 0, 0)),
        out_shape=jax.ShapeDtypeStruct((_E, _IN, _HID), jnp.float32),
    )(W, W_edge)


def _out_body(agg_ref, m_ref, bias_ref, o_ref):
    e = pl.program_id(1)
    part = jnp.dot(agg_ref[0, 0], m_ref[0], preferred_element_type=jnp.float32)

    @pl.when(e == 0)
    def _():
        o_ref[0] = part

    @pl.when(e != 0)
    def _():
        o_ref[0] += part

    @pl.when(e == _E - 1)
    def _():
        o_ref[0] = jnp.maximum(o_ref[0] * (1.0 / _K) + bias_ref[...], 0.0)


def _finalize(agg, M, bias2d):
    nb = agg.shape[0]
    return pl.pallas_call(
        _out_body,
        grid=(nb, _E),
        in_specs=[pl.BlockSpec((1, 1, _N, _IN), lambda b, e: (b, e, 0, 0)),
                  pl.BlockSpec((1, _IN, _HID), lambda b, e: (e, 0, 0)),
                  pl.BlockSpec((1, _HID), lambda b, e: (0, 0))],
        out_specs=pl.BlockSpec((1, _N, _HID), lambda b, e: (b, 0, 0)),
        out_shape=jax.ShapeDtypeStruct((nb, _N, _HID), jnp.float32),
        compiler_params=pltpu.CompilerParams(
            dimension_semantics=("arbitrary", "arbitrary")),
    )(agg, M, bias2d)


def kernel(nodes_embed, node_nei, node_edge, W, W_edge, bias):
    x_flat = nodes_embed.reshape(_NODES, _IN)
    b_idx = jnp.arange(_B, dtype=jnp.int32)[:, None, None]
    gidx = (node_nei + b_idx * _N).reshape(_NODES * _K)
    n_local = (jnp.arange(_N, dtype=jnp.int32) % _NPJ)[None, :, None]
    drow = (node_edge * _NPJ + n_local).reshape(_NODES * _K)
    basev = (drow[:, None] * _IN
             + jnp.arange(16, dtype=jnp.int32)[None, :]).reshape(-1)
    M = _fuse_weights(W, W_edge)
    bias2 = bias.reshape(1, _HID)
    hk = _NH * _K
    aggs = [_sc_agg(x_flat, gidx[h * hk:(h + 1) * hk],
                    basev[h * hk * 16:(h + 1) * hk * 16])
            for h in range(2)]
    outs = [_finalize(a.reshape(2, _E, _N, _IN), M, bias2) for a in aggs]
    return jnp.concatenate(outs, axis=0)
